# baseline (speedup 1.0000x reference)
import jax
import jax.numpy as jnp
from jax import lax
from jax.experimental import pallas as pl
from jax.experimental.pallas import tpu as pltpu
from jax.experimental.pallas import tpu_sc as plsc

N = 10000
E = 160000
R = 2
EMB = 128
HID = 64
NB = 2
IN_DIM = 3

NPAD = 10240
BLK = 1024
NBLK = NPAD // BLK
NCK = 176
MAXEP = NCK * BLK
NCORE = 2
NSUB = 16
NW = NCORE * NSUB
EPW = MAXEP // NW
C = 128
NCHUNK = EPW // C
GRID = NPAD // BLK

_f32 = jnp.float32


def _make_edge_tanh():
    mesh = plsc.VectorSubcoreMesh(core_axis_name="c", subcore_axis_name="s")
    scratch = [
        pltpu.VMEM((EPW,), jnp.int32),
        pltpu.VMEM((EPW,), jnp.int32),
        pltpu.VMEM((C, EMB), _f32),
        pltpu.VMEM((C, EMB), _f32),
        pltpu.VMEM((C, EMB), _f32),
        pltpu.VMEM((C, EMB), _f32),
        pltpu.SemaphoreType.DMA,
        pltpu.SemaphoreType.DMA,
        pltpu.SemaphoreType.DMA,
        pltpu.SemaphoreType.DMA,
    ]

    def body(src_ref, dst_ref, a_hbm, b_hbm, t_out,
             isrc, idst, a0, b0, a1, b1, s0, s1, s2, s3):
        cid = lax.axis_index("c")
        sid = lax.axis_index("s")
        wid = cid * NSUB + sid
        base0 = wid * EPW
        pltpu.sync_copy(src_ref.at[pl.ds(base0, EPW)], isrc)
        pltpu.sync_copy(dst_ref.at[pl.ds(base0, EPW)], idst)
        bufs = ((a0, b0, s0, s1), (a1, b1, s2, s3))

        @pl.loop(0, NCHUNK, step=2)
        def pair_body(c0):
            handles = []
            for k in range(2):
                ab, bb, sa, sb = bufs[k]
                off = (c0 + k) * C
                idxs = isrc.at[pl.ds(off, C)]
                idxd = idst.at[pl.ds(off, C)]
                handles.append((pltpu.async_copy(a_hbm.at[idxs], ab, sa),
                                pltpu.async_copy(b_hbm.at[idxd], bb, sb)))
            for k in range(2):
                ab, bb, sa, sb = bufs[k]
                ha, hb = handles[k]
                ha.wait()
                hb.wait()

                def ew_body(r, carry2):
                    for j in range(EMB // 16):
                        sl = pl.ds(j * 16, 16)
                        ab[r, sl] = ab[r, sl] + bb[r, sl]
                    return carry2

                lax.fori_loop(0, C, ew_body, 0)
                pltpu.sync_copy(ab, t_out.at[pl.ds(base0 + (c0 + k) * C, C)])

    return pl.kernel(body, out_type=jax.ShapeDtypeStruct((MAXEP, EMB), _f32),
                     mesh=mesh, scratch_types=scratch)


def _segsum_body(blk_ref, dst_ref, t_ref, agg_ref, deg_ref):
    i = pl.program_id(0)
    blk = blk_ref[i]
    prev = blk_ref[jnp.maximum(i - 1, 0)]
    first = jnp.logical_or(i == 0, blk != prev)
    nodeval = blk * BLK + lax.broadcasted_iota(jnp.int32, (BLK, 1), 0)
    d = dst_ref[0, 0, :].reshape(1, BLK)
    p = nodeval == d
    pb = p.astype(jnp.bfloat16)
    contrib = jnp.dot(pb, jnp.tanh(t_ref[...]).astype(jnp.bfloat16),
                      preferred_element_type=_f32)
    degc = jnp.sum(p.astype(_f32), axis=1, keepdims=True)
    degc8 = jnp.concatenate([degc, jnp.zeros((BLK, 7), _f32)], axis=1)

    @pl.when(first)
    def _():
        agg_ref[...] = jnp.zeros_like(agg_ref)
        deg_ref[...] = jnp.zeros_like(deg_ref)

    agg_ref[...] += contrib
    deg_ref[...] += degc8


def _segsum_call(blkid, dstid3, t):
    return pl.pallas_call(
        _segsum_body,
        grid_spec=pltpu.PrefetchScalarGridSpec(
            num_scalar_prefetch=1,
            grid=(NCK,),
            in_specs=[pl.BlockSpec((1, 1, BLK), lambda i, b: (i, 0, 0)),
                      pl.BlockSpec((BLK, EMB), lambda i, b: (i, 0))],
            out_specs=[pl.BlockSpec((BLK, EMB), lambda i, b: (b[i], 0)),
                       pl.BlockSpec((BLK, 8), lambda i, b: (b[i], 0))],
        ),
        out_shape=[jax.ShapeDtypeStruct((NPAD, EMB), _f32),
                   jax.ShapeDtypeStruct((NPAD, 8), _f32)],
    )(blkid, dstid3, t)


def _dot(a, b):
    return jnp.dot(a, b, preferred_element_type=_f32)


def _row_spec(width):
    return pl.BlockSpec((BLK, width), lambda i: (i, 0))


def _full_spec(shape):
    nd = len(shape)
    return pl.BlockSpec(shape, lambda i: (0,) * nd)


def _embed(x_ref, ew_ref, eb_ref):
    pre = (x_ref[:, 0:1] * ew_ref[0:1, :] + x_ref[:, 1:2] * ew_ref[1:2, :]
           + eb_ref[0:1, :])
    return jnp.tanh(pre)


def _tables(h, ca_ref, cbw_ref, cbb_ref, a_ref, b_ref):
    a_ref[...] = _dot(h, ca_ref[...])
    b_ref[...] = _dot(h, cbw_ref[...]) + cbb_ref[0:1, :]


def _update(h, s_ref, d_ref, w2_ref, b2_ref, u1a_ref, u1b_ref, ub1_ref,
            u2_ref, ub2_ref):
    agg = _dot(s_ref[...], w2_ref[...]) + d_ref[:, 0:1] * b2_ref[0:1, :]
    u = jnp.tanh(_dot(h, u1a_ref[...]) + _dot(agg, u1b_ref[...])
                 + ub1_ref[0:1, :])
    return h + _dot(u, u2_ref[...]) + ub2_ref[0:1, :]


def _dense_first_body(x_ref, ew_ref, eb_ref, ca_ref, cbw_ref, cbb_ref,
                      h_ref, a_ref, b_ref):
    h = _embed(x_ref, ew_ref, eb_ref)
    h_ref[...] = h
    _tables(h, ca_ref, cbw_ref, cbb_ref, a_ref, b_ref)


def _dense_mid_body(h_ref, s_ref, d_ref, w2_ref, b2_ref, u1a_ref, u1b_ref,
                    ub1_ref, u2_ref, ub2_ref, ca_ref, cbw_ref, cbb_ref,
                    hn_ref, a_ref, b_ref):
    hn = _update(h_ref[...], s_ref, d_ref, w2_ref, b2_ref, u1a_ref, u1b_ref,
                 ub1_ref, u2_ref, ub2_ref)
    hn_ref[...] = hn
    _tables(hn, ca_ref, cbw_ref, cbb_ref, a_ref, b_ref)


def _dense_end_body(h_ref, s_ref, d_ref, w2_ref, b2_ref, u1a_ref, u1b_ref,
                    ub1_ref, u2_ref, ub2_ref, o1_ref, ob1_ref, o2_ref,
                    ob2_ref, xo_ref, lp_ref, xon_ref, lpn_ref):
    hn = _update(h_ref[...], s_ref, d_ref, w2_ref, b2_ref, u1a_ref, u1b_ref,
                 ub1_ref, u2_ref, ub2_ref)
    o = _dot(jnp.tanh(_dot(hn, o1_ref[...]) + ob1_ref[0:1, :]), o2_ref[...]) \
        + ob2_ref[0:1, :]
    s8 = o[:, 0:8]
    t8 = o[:, 128:136]
    xon_ref[...] = xo_ref[...] * jnp.exp(s8) + t8
    lpn_ref[...] = lp_ref[...] - jnp.sum(s8, axis=1, keepdims=True)


def _dense_first_call(xh, m):
    return pl.pallas_call(
        _dense_first_body,
        grid=(GRID,),
        in_specs=[_row_spec(8), _full_spec((8, 256)), _full_spec((8, 256)),
                  _full_spec((256, EMB)), _full_spec((256, EMB)),
                  _full_spec((8, EMB))],
        out_specs=[_row_spec(256), _row_spec(EMB), _row_spec(EMB)],
        out_shape=[jax.ShapeDtypeStruct((NPAD, 256), _f32),
                   jax.ShapeDtypeStruct((NPAD, EMB), _f32),
                   jax.ShapeDtypeStruct((NPAD, EMB), _f32)],
    )(xh, m['EW'], m['EB'], m['CA0'], m['CB0'], m['cb0'])


_SD_SPECS = [_row_spec(EMB), _row_spec(8)]
_MLP_SPECS = [_full_spec((EMB, 256)), _full_spec((8, 256)),
              _full_spec((256, EMB)), _full_spec((256, EMB)),
              _full_spec((8, EMB)), _full_spec((EMB, 256)),
              _full_spec((8, 256))]


def _dense_mid_call(h, s, d, m, k):
    return pl.pallas_call(
        _dense_mid_body,
        grid=(GRID,),
        in_specs=[_row_spec(256)] + _SD_SPECS + _MLP_SPECS
                 + [_full_spec((256, EMB)), _full_spec((256, EMB)),
                    _full_spec((8, EMB))],
        out_specs=[_row_spec(256), _row_spec(EMB), _row_spec(EMB)],
        out_shape=[jax.ShapeDtypeStruct((NPAD, 256), _f32),
                   jax.ShapeDtypeStruct((NPAD, EMB), _f32),
                   jax.ShapeDtypeStruct((NPAD, EMB), _f32)],
    )(h, s, d, m[f'W2{k}'], m[f'b2{k}'], m[f'U1a{k}'], m[f'U1b{k}'],
      m[f'ub1{k}'], m[f'U2{k}'], m[f'ub2{k}'], m[f'CA{k + 1}'],
      m[f'CB{k + 1}'], m[f'cb{k + 1}'])


def _dense_end_call(h, s, d, m, k, xo, lp):
    return pl.pallas_call(
        _dense_end_body,
        grid=(GRID,),
        in_specs=[_row_spec(256)] + _SD_SPECS + _MLP_SPECS
                 + [_full_spec((256, EMB)), _full_spec((8, EMB)),
                    _full_spec((EMB, 256)), _full_spec((8, 256)),
                    _row_spec(8), _row_spec(8)],
        out_specs=[_row_spec(8), _row_spec(8)],
        out_shape=[jax.ShapeDtypeStruct((NPAD, 8), _f32),
                   jax.ShapeDtypeStruct((NPAD, 8), _f32)],
    )(h, s, d, m[f'W2{k}'], m[f'b2{k}'], m[f'U1a{k}'], m[f'U1b{k}'],
      m[f'ub1{k}'], m[f'U2{k}'], m[f'ub2{k}'], m['O1'], m['ob1'], m['O2'],
      m['ob2'], xo, lp)


def _bd(a, b):
    ra, ca = a.shape
    rb, cb = b.shape
    top = jnp.concatenate([a, jnp.zeros((ra, cb), _f32)], axis=1)
    bot = jnp.concatenate([jnp.zeros((rb, ca), _f32), b], axis=1)
    return jnp.concatenate([top, bot], axis=0)


def _row8(v, width):
    return jnp.concatenate([v[None, :], jnp.zeros((7, width), _f32)], axis=0)


def _stage_mats(ps, pt, din, dout):
    m = {}
    ew = jnp.concatenate([ps['embed_W'], pt['embed_W']], axis=1)
    m['EW'] = jnp.concatenate([ew, jnp.zeros((8 - din, 256), _f32)], axis=0)
    m['EB'] = _row8(jnp.concatenate([ps['embed_b'], pt['embed_b']]), 256)
    for k in range(NB):
        bs, bt = ps['blocks'][k], pt['blocks'][k]
        m[f'CA{k}'] = _bd(bs['msg_W1'][:EMB], bt['msg_W1'][:EMB])
        m[f'CB{k}'] = _bd(bs['msg_W1'][EMB:], bt['msg_W1'][EMB:])
        m[f'cb{k}'] = _row8(
            jnp.concatenate([bs['msg_b1'], bt['msg_b1']]), EMB)
        m[f'W2{k}'] = _bd(bs['msg_W2'], bt['msg_W2'])
        m[f'b2{k}'] = _row8(
            jnp.concatenate([bs['msg_b2'], bt['msg_b2']]), 256)
        m[f'U1a{k}'] = _bd(bs['upd_W1'][:EMB], bt['upd_W1'][:EMB])
        m[f'U1b{k}'] = _bd(bs['upd_W1'][EMB:], bt['upd_W1'][EMB:])
        m[f'ub1{k}'] = _row8(
            jnp.concatenate([bs['upd_b1'], bt['upd_b1']]), EMB)
        m[f'U2{k}'] = _bd(bs['upd_W2'], bt['upd_W2'])
        m[f'ub2{k}'] = _row8(
            jnp.concatenate([bs['upd_b2'], bt['upd_b2']]), 256)
    m['O1'] = _bd(ps['out_W1'], pt['out_W1'])
    m['ob1'] = _row8(jnp.concatenate([ps['out_b1'], pt['out_b1']]), EMB)
    o2 = jnp.zeros((EMB, 256), _f32)
    o2 = o2.at[0:HID, 0:dout].set(ps['out_W2'])
    o2 = o2.at[HID:EMB, 128:128 + dout].set(pt['out_W2'])
    m['O2'] = o2
    ob2 = jnp.zeros((256,), _f32)
    ob2 = ob2.at[0:dout].set(ps['out_b2'])
    ob2 = ob2.at[128:128 + dout].set(pt['out_b2'])
    m['ob2'] = _row8(ob2, 256)
    return m


def _pad_cols(a, width):
    out = jnp.zeros((NPAD, width), _f32)
    return out.at[:a.shape[0], :a.shape[1]].set(a)


def _bucket_edges(context):
    src0 = context[0].astype(jnp.int32)
    dst0 = context[1].astype(jnp.int32)
    ordi = jnp.argsort(dst0)
    src_s = src0[ordi]
    dst_s = dst0[ordi]
    bounds = jnp.searchsorted(
        dst_s, jnp.arange(0, NPAD + 1, BLK, dtype=jnp.int32)).astype(jnp.int32)
    cnt = bounds[1:] - bounds[:-1]
    kb = jnp.maximum((cnt + (BLK - 1)) // BLK, 1)
    cend = jnp.cumsum(kb).astype(jnp.int32)
    cstart = cend - kb
    blk_e = dst_s // BLK
    local = jnp.arange(E, dtype=jnp.int32) - bounds[blk_e]
    pos = cstart[blk_e] * BLK + local
    src_pad = jnp.zeros((MAXEP,), jnp.int32).at[pos].set(src_s)
    dstid = jnp.full((MAXEP,), -1, jnp.int32).at[pos].set(dst_s)
    dst_g = jnp.where(dstid < 0, NPAD - 1, dstid)
    blkid = jnp.minimum(
        jnp.searchsorted(cend, jnp.arange(NCK, dtype=jnp.int32), side='right'),
        NBLK - 1).astype(jnp.int32)
    return src_pad, dst_g, dstid.reshape(NCK, 1, BLK), blkid


def kernel(x, context, logpx, params):
    x1 = _pad_cols(x[:, :1], 8)
    x2 = _pad_cols(x[:, 1:], 8)
    lp = _pad_cols(logpx, 8)
    src_pad, dst_g, dstid3, blkid = _bucket_edges(context)
    edge_tanh = _make_edge_tanh()

    stages = []
    for i in range(R):
        stages.append((params['s0'][i], params['t0'][i], 1, 2))
        stages.append((params['s1'][i], params['t1'][i], 2, 1))

    for si, (ps, pt, din, dout) in enumerate(stages):
        m = _stage_mats(ps, pt, din, dout)
        xin, xo = (x1, x2) if din == 1 else (x2, x1)
        h, a, b = _dense_first_call(xin, m)
        t = edge_tanh(src_pad, dst_g, a, b)
        s, deg = _segsum_call(blkid, dstid3, t)
        h, a, b = _dense_mid_call(h, s, deg, m, 0)
        t = edge_tanh(src_pad, dst_g, a, b)
        s, deg = _segsum_call(blkid, dstid3, t)
        xo_new, lp = _dense_end_call(h, s, deg, m, NB - 1, xo, lp)
        if din == 1:
            x2 = xo_new
        else:
            x1 = xo_new

    z = jnp.concatenate([x1[:N, 0:1], x2[:N, 0:2]], axis=1)
    return (z, lp[:N, 0:1])


# speedup vs baseline: 1.0362x; 1.0362x over previous
import jax
import jax.numpy as jnp
from jax import lax
from jax.experimental import pallas as pl
from jax.experimental.pallas import tpu as pltpu
from jax.experimental.pallas import tpu_sc as plsc

N = 10000
E = 160000
R = 2
EMB = 128
HID = 64
NB = 2
IN_DIM = 3

NPAD = 10240
BLK = 1024
NBLK = NPAD // BLK
NCK = 176
MAXEP = NCK * BLK
NCORE = 2
NSUB = 16
NW = NCORE * NSUB
EPW = MAXEP // NW
C = 128
NCHUNK = EPW // C
GRID = NPAD // BLK

_f32 = jnp.float32


def _make_edge_tanh():
    mesh = plsc.VectorSubcoreMesh(core_axis_name="c", subcore_axis_name="s")
    scratch = [
        pltpu.VMEM((EPW,), jnp.int32),
        pltpu.VMEM((C, EMB), _f32),
        pltpu.VMEM((C, EMB), _f32),
        pltpu.SemaphoreType.DMA,
        pltpu.SemaphoreType.DMA,
    ]

    def body(src_ref, a_hbm, t_out, isrc, a0, a1, s0, s1):
        cid = lax.axis_index("c")
        sid = lax.axis_index("s")
        wid = cid * NSUB + sid
        base0 = wid * EPW
        pltpu.sync_copy(src_ref.at[pl.ds(base0, EPW)], isrc)
        bufs = ((a0, s0), (a1, s1))

        @pl.loop(0, NCHUNK, step=2)
        def pair_body(c0):
            handles = []
            for k in range(2):
                ab, sa = bufs[k]
                idxs = isrc.at[pl.ds((c0 + k) * C, C)]
                handles.append(pltpu.async_copy(a_hbm.at[idxs], ab, sa))
            for k in range(2):
                ab, sa = bufs[k]
                handles[k].wait()
                pltpu.sync_copy(ab, t_out.at[pl.ds(base0 + (c0 + k) * C, C)])

    return pl.kernel(body, out_type=jax.ShapeDtypeStruct((MAXEP, EMB), _f32),
                     mesh=mesh, scratch_types=scratch)


def _segsum_body(blk_ref, dst_ref, t_ref, b_ref, agg_ref, deg_ref):
    i = pl.program_id(0)
    blk = blk_ref[i]
    prev = blk_ref[jnp.maximum(i - 1, 0)]
    first = jnp.logical_or(i == 0, blk != prev)
    nodeval = blk * BLK + lax.broadcasted_iota(jnp.int32, (BLK, 1), 0)
    d = dst_ref[0, 0, :].reshape(1, BLK)
    p = nodeval == d
    pt = (d.reshape(BLK, 1) == nodeval.reshape(1, BLK)).astype(_f32)
    bg = jnp.dot(pt, b_ref[...], preferred_element_type=_f32)
    tv = jnp.tanh(t_ref[...] + bg)
    pb = p.astype(jnp.bfloat16)
    contrib = jnp.dot(pb, tv.astype(jnp.bfloat16), preferred_element_type=_f32)
    degc = jnp.sum(p.astype(_f32), axis=1, keepdims=True)
    degc8 = jnp.concatenate([degc, jnp.zeros((BLK, 7), _f32)], axis=1)

    @pl.when(first)
    def _():
        agg_ref[...] = jnp.zeros_like(agg_ref)
        deg_ref[...] = jnp.zeros_like(deg_ref)

    agg_ref[...] += contrib
    deg_ref[...] += degc8


def _segsum_call(blkid, dstid3, t, bmat):
    return pl.pallas_call(
        _segsum_body,
        grid_spec=pltpu.PrefetchScalarGridSpec(
            num_scalar_prefetch=1,
            grid=(NCK,),
            in_specs=[pl.BlockSpec((1, 1, BLK), lambda i, b: (i, 0, 0)),
                      pl.BlockSpec((BLK, EMB), lambda i, b: (i, 0)),
                      pl.BlockSpec((BLK, EMB), lambda i, b: (b[i], 0))],
            out_specs=[pl.BlockSpec((BLK, EMB), lambda i, b: (b[i], 0)),
                       pl.BlockSpec((BLK, 8), lambda i, b: (b[i], 0))],
        ),
        out_shape=[jax.ShapeDtypeStruct((NPAD, EMB), _f32),
                   jax.ShapeDtypeStruct((NPAD, 8), _f32)],
    )(blkid, dstid3, t, bmat)


def _dot(a, b):
    return jnp.dot(a, b, preferred_element_type=_f32)


def _row_spec(width):
    return pl.BlockSpec((BLK, width), lambda i: (i, 0))


def _full_spec(shape):
    nd = len(shape)
    return pl.BlockSpec(shape, lambda i: (0,) * nd)


def _embed(x_ref, ew_ref, eb_ref):
    pre = (x_ref[:, 0:1] * ew_ref[0:1, :] + x_ref[:, 1:2] * ew_ref[1:2, :]
           + eb_ref[0:1, :])
    return jnp.tanh(pre)


def _tables(h, ca_ref, cbw_ref, cbb_ref, a_ref, b_ref):
    a_ref[...] = _dot(h, ca_ref[...])
    b_ref[...] = _dot(h, cbw_ref[...]) + cbb_ref[0:1, :]


def _update(h, s_ref, d_ref, w2_ref, b2_ref, u1a_ref, u1b_ref, ub1_ref,
            u2_ref, ub2_ref):
    agg = _dot(s_ref[...], w2_ref[...]) + d_ref[:, 0:1] * b2_ref[0:1, :]
    u = jnp.tanh(_dot(h, u1a_ref[...]) + _dot(agg, u1b_ref[...])
                 + ub1_ref[0:1, :])
    return h + _dot(u, u2_ref[...]) + ub2_ref[0:1, :]


def _dense_first_body(x_ref, ew_ref, eb_ref, ca_ref, cbw_ref, cbb_ref,
                      h_ref, a_ref, b_ref):
    h = _embed(x_ref, ew_ref, eb_ref)
    h_ref[...] = h
    _tables(h, ca_ref, cbw_ref, cbb_ref, a_ref, b_ref)


def _dense_mid_body(h_ref, s_ref, d_ref, w2_ref, b2_ref, u1a_ref, u1b_ref,
                    ub1_ref, u2_ref, ub2_ref, ca_ref, cbw_ref, cbb_ref,
                    hn_ref, a_ref, b_ref):
    hn = _update(h_ref[...], s_ref, d_ref, w2_ref, b2_ref, u1a_ref, u1b_ref,
                 ub1_ref, u2_ref, ub2_ref)
    hn_ref[...] = hn
    _tables(hn, ca_ref, cbw_ref, cbb_ref, a_ref, b_ref)


def _dense_end_body(h_ref, s_ref, d_ref, w2_ref, b2_ref, u1a_ref, u1b_ref,
                    ub1_ref, u2_ref, ub2_ref, o1_ref, ob1_ref, o2_ref,
                    ob2_ref, xo_ref, lp_ref, xon_ref, lpn_ref):
    hn = _update(h_ref[...], s_ref, d_ref, w2_ref, b2_ref, u1a_ref, u1b_ref,
                 ub1_ref, u2_ref, ub2_ref)
    o = _dot(jnp.tanh(_dot(hn, o1_ref[...]) + ob1_ref[0:1, :]), o2_ref[...]) \
        + ob2_ref[0:1, :]
    s8 = o[:, 0:8]
    t8 = o[:, 128:136]
    xon_ref[...] = xo_ref[...] * jnp.exp(s8) + t8
    lpn_ref[...] = lp_ref[...] - jnp.sum(s8, axis=1, keepdims=True)


def _dense_first_call(xh, m):
    return pl.pallas_call(
        _dense_first_body,
        grid=(GRID,),
        in_specs=[_row_spec(8), _full_spec((8, 256)), _full_spec((8, 256)),
                  _full_spec((256, EMB)), _full_spec((256, EMB)),
                  _full_spec((8, EMB))],
        out_specs=[_row_spec(256), _row_spec(EMB), _row_spec(EMB)],
        out_shape=[jax.ShapeDtypeStruct((NPAD, 256), _f32),
                   jax.ShapeDtypeStruct((NPAD, EMB), _f32),
                   jax.ShapeDtypeStruct((NPAD, EMB), _f32)],
    )(xh, m['EW'], m['EB'], m['CA0'], m['CB0'], m['cb0'])


_SD_SPECS = [_row_spec(EMB), _row_spec(8)]
_MLP_SPECS = [_full_spec((EMB, 256)), _full_spec((8, 256)),
              _full_spec((256, EMB)), _full_spec((256, EMB)),
              _full_spec((8, EMB)), _full_spec((EMB, 256)),
              _full_spec((8, 256))]


def _dense_mid_call(h, s, d, m, k):
    return pl.pallas_call(
        _dense_mid_body,
        grid=(GRID,),
        in_specs=[_row_spec(256)] + _SD_SPECS + _MLP_SPECS
                 + [_full_spec((256, EMB)), _full_spec((256, EMB)),
                    _full_spec((8, EMB))],
        out_specs=[_row_spec(256), _row_spec(EMB), _row_spec(EMB)],
        out_shape=[jax.ShapeDtypeStruct((NPAD, 256), _f32),
                   jax.ShapeDtypeStruct((NPAD, EMB), _f32),
                   jax.ShapeDtypeStruct((NPAD, EMB), _f32)],
    )(h, s, d, m[f'W2{k}'], m[f'b2{k}'], m[f'U1a{k}'], m[f'U1b{k}'],
      m[f'ub1{k}'], m[f'U2{k}'], m[f'ub2{k}'], m[f'CA{k + 1}'],
      m[f'CB{k + 1}'], m[f'cb{k + 1}'])


def _dense_end_call(h, s, d, m, k, xo, lp):
    return pl.pallas_call(
        _dense_end_body,
        grid=(GRID,),
        in_specs=[_row_spec(256)] + _SD_SPECS + _MLP_SPECS
                 + [_full_spec((256, EMB)), _full_spec((8, EMB)),
                    _full_spec((EMB, 256)), _full_spec((8, 256)),
                    _row_spec(8), _row_spec(8)],
        out_specs=[_row_spec(8), _row_spec(8)],
        out_shape=[jax.ShapeDtypeStruct((NPAD, 8), _f32),
                   jax.ShapeDtypeStruct((NPAD, 8), _f32)],
    )(h, s, d, m[f'W2{k}'], m[f'b2{k}'], m[f'U1a{k}'], m[f'U1b{k}'],
      m[f'ub1{k}'], m[f'U2{k}'], m[f'ub2{k}'], m['O1'], m['ob1'], m['O2'],
      m['ob2'], xo, lp)


def _bd(a, b):
    ra, ca = a.shape
    rb, cb = b.shape
    top = jnp.concatenate([a, jnp.zeros((ra, cb), _f32)], axis=1)
    bot = jnp.concatenate([jnp.zeros((rb, ca), _f32), b], axis=1)
    return jnp.concatenate([top, bot], axis=0)


def _row8(v, width):
    return jnp.concatenate([v[None, :], jnp.zeros((7, width), _f32)], axis=0)


def _stage_mats(ps, pt, din, dout):
    m = {}
    ew = jnp.concatenate([ps['embed_W'], pt['embed_W']], axis=1)
    m['EW'] = jnp.concatenate([ew, jnp.zeros((8 - din, 256), _f32)], axis=0)
    m['EB'] = _row8(jnp.concatenate([ps['embed_b'], pt['embed_b']]), 256)
    for k in range(NB):
        bs, bt = ps['blocks'][k], pt['blocks'][k]
        m[f'CA{k}'] = _bd(bs['msg_W1'][:EMB], bt['msg_W1'][:EMB])
        m[f'CB{k}'] = _bd(bs['msg_W1'][EMB:], bt['msg_W1'][EMB:])
        m[f'cb{k}'] = _row8(
            jnp.concatenate([bs['msg_b1'], bt['msg_b1']]), EMB)
        m[f'W2{k}'] = _bd(bs['msg_W2'], bt['msg_W2'])
        m[f'b2{k}'] = _row8(
            jnp.concatenate([bs['msg_b2'], bt['msg_b2']]), 256)
        m[f'U1a{k}'] = _bd(bs['upd_W1'][:EMB], bt['upd_W1'][:EMB])
        m[f'U1b{k}'] = _bd(bs['upd_W1'][EMB:], bt['upd_W1'][EMB:])
        m[f'ub1{k}'] = _row8(
            jnp.concatenate([bs['upd_b1'], bt['upd_b1']]), EMB)
        m[f'U2{k}'] = _bd(bs['upd_W2'], bt['upd_W2'])
        m[f'ub2{k}'] = _row8(
            jnp.concatenate([bs['upd_b2'], bt['upd_b2']]), 256)
    m['O1'] = _bd(ps['out_W1'], pt['out_W1'])
    m['ob1'] = _row8(jnp.concatenate([ps['out_b1'], pt['out_b1']]), EMB)
    o2 = jnp.zeros((EMB, 256), _f32)
    o2 = o2.at[0:HID, 0:dout].set(ps['out_W2'])
    o2 = o2.at[HID:EMB, 128:128 + dout].set(pt['out_W2'])
    m['O2'] = o2
    ob2 = jnp.zeros((256,), _f32)
    ob2 = ob2.at[0:dout].set(ps['out_b2'])
    ob2 = ob2.at[128:128 + dout].set(pt['out_b2'])
    m['ob2'] = _row8(ob2, 256)
    return m


def _pad_cols(a, width):
    out = jnp.zeros((NPAD, width), _f32)
    return out.at[:a.shape[0], :a.shape[1]].set(a)


def _bucket_edges(context):
    src0 = context[0].astype(jnp.int32)
    dst0 = context[1].astype(jnp.int32)
    ordi = jnp.argsort(dst0)
    src_s = src0[ordi]
    dst_s = dst0[ordi]
    bounds = jnp.searchsorted(
        dst_s, jnp.arange(0, NPAD + 1, BLK, dtype=jnp.int32)).astype(jnp.int32)
    cnt = bounds[1:] - bounds[:-1]
    kb = jnp.maximum((cnt + (BLK - 1)) // BLK, 1)
    cend = jnp.cumsum(kb).astype(jnp.int32)
    cstart = cend - kb
    blk_e = dst_s // BLK
    local = jnp.arange(E, dtype=jnp.int32) - bounds[blk_e]
    pos = cstart[blk_e] * BLK + local
    src_pad = jnp.zeros((MAXEP,), jnp.int32).at[pos].set(src_s)
    dstid = jnp.full((MAXEP,), -1, jnp.int32).at[pos].set(dst_s)
    dst_g = jnp.where(dstid < 0, NPAD - 1, dstid)
    blkid = jnp.minimum(
        jnp.searchsorted(cend, jnp.arange(NCK, dtype=jnp.int32), side='right'),
        NBLK - 1).astype(jnp.int32)
    return src_pad, dst_g, dstid.reshape(NCK, 1, BLK), blkid


def kernel(x, context, logpx, params):
    x1 = _pad_cols(x[:, :1], 8)
    x2 = _pad_cols(x[:, 1:], 8)
    lp = _pad_cols(logpx, 8)
    src_pad, dst_g, dstid3, blkid = _bucket_edges(context)
    edge_tanh = _make_edge_tanh()

    stages = []
    for i in range(R):
        stages.append((params['s0'][i], params['t0'][i], 1, 2))
        stages.append((params['s1'][i], params['t1'][i], 2, 1))

    for si, (ps, pt, din, dout) in enumerate(stages):
        m = _stage_mats(ps, pt, din, dout)
        xin, xo = (x1, x2) if din == 1 else (x2, x1)
        h, a, b = _dense_first_call(xin, m)
        t = edge_tanh(src_pad, a)
        s, deg = _segsum_call(blkid, dstid3, t, b)
        h, a, b = _dense_mid_call(h, s, deg, m, 0)
        t = edge_tanh(src_pad, a)
        s, deg = _segsum_call(blkid, dstid3, t, b)
        xo_new, lp = _dense_end_call(h, s, deg, m, NB - 1, xo, lp)
        if din == 1:
            x2 = xo_new
        else:
            x1 = xo_new

    z = jnp.concatenate([x1[:N, 0:1], x2[:N, 0:2]], axis=1)
    return (z, lp[:N, 0:1])
